# baseline (speedup 1.0000x reference)
import jax
import jax.numpy as jnp
from jax.experimental import pallas as pl

N = 10000
D = 128
FH1 = 64
FH2 = 32
GH1 = 32
GH2 = 16
LAT = FH2 + GH2
EPS = 1e-3

BM = 200


def _bn(x, g, b, rm, rv):
    return (x - rm) / jnp.sqrt(rv + EPS) * g + b


def _elu(x):
    return jnp.where(x > 0, x, jnp.exp(x) - 1.0)


def _encoder_kernel(x_ref, W1_ref, b1_ref, g1_ref, be1_ref, rm1_ref, rv1_ref,
                    W2_ref, b2_ref, g2_ref, be2_ref, rm2_ref, rv2_ref,
                    Wg1_ref, feat_ref, s1_ref):
    h = jnp.dot(x_ref[...], W1_ref[...], preferred_element_type=jnp.float32)
    h = _elu(_bn(h + b1_ref[...], g1_ref[...], be1_ref[...],
                       rm1_ref[...], rv1_ref[...]))
    f = jnp.dot(h, W2_ref[...], preferred_element_type=jnp.float32)
    f = _elu(_bn(f + b2_ref[...], g2_ref[...], be2_ref[...],
                       rm2_ref[...], rv2_ref[...]))
    feat_ref[...] = f
    s1_ref[...] = jnp.dot(f, Wg1_ref[...], preferred_element_type=jnp.float32)


def _spmm1_kernel(adj_ref, s1_ref, Wg23_ref, s2_ref):
    h1 = jnp.dot(adj_ref[...], s1_ref[...], preferred_element_type=jnp.float32)
    s2_ref[...] = jnp.dot(h1, Wg23_ref[...], preferred_element_type=jnp.float32)


def _spmm2_dec_kernel(adj_ref, s2_ref, feat_ref,
                      Wd1_ref, bd1_ref, gd1_ref, bed1_ref, rmd1_ref, rvd1_ref,
                      Wd2_ref, bd2_ref, gd2_ref, bed2_ref, rmd2_ref, rvd2_ref,
                      mu_ref, ls_ref, z_ref, dec_ref):
    out2 = jnp.dot(adj_ref[...], s2_ref[...], preferred_element_type=jnp.float32)
    mu = out2[:, :GH2]
    mu_ref[...] = mu
    ls_ref[...] = out2[:, GH2:]
    z = jnp.concatenate([feat_ref[...], mu], axis=1)
    z_ref[...] = z
    d = jnp.dot(z, Wd1_ref[...], preferred_element_type=jnp.float32)
    d = _elu(_bn(d + bd1_ref[...], gd1_ref[...], bed1_ref[...],
                       rmd1_ref[...], rvd1_ref[...]))
    dec = jnp.dot(d, Wd2_ref[...], preferred_element_type=jnp.float32)
    dec_ref[...] = jax.nn.relu(_bn(dec + bd2_ref[...], gd2_ref[...], bed2_ref[...],
                                   rmd2_ref[...], rvd2_ref[...]))


def _row(v):
    return v.reshape(1, -1)


def kernel(x, adj, W1, b1, g1, be1, rm1, rv1, W2, b2, g2, be2, rm2, rv2,
           Wg1, Wg2, Wg3,
           Wd1, bd1, gd1, bed1, rmd1, rvd1,
           Wd2, bd2, gd2, bed2, rmd2, rvd2):
    f32 = jnp.float32

    full = lambda s: pl.BlockSpec(s, lambda: (0, 0))
    feat_x, s1 = pl.pallas_call(
        _encoder_kernel,
        grid=(),
        in_specs=[full((N, D)),
                  full((D, FH1))] + [full((1, FH1))] * 5 +
                 [full((FH1, FH2))] + [full((1, FH2))] * 5 +
                 [full((FH2, GH1))],
        out_specs=[full((N, FH2)), full((N, GH1))],
        out_shape=[jax.ShapeDtypeStruct((N, FH2), f32),
                   jax.ShapeDtypeStruct((N, GH1), f32)],
    )(x, W1, _row(b1), _row(g1), _row(be1), _row(rm1), _row(rv1),
      W2, _row(b2), _row(g2), _row(be2), _row(rm2), _row(rv2), Wg1)

    Wg23 = jnp.concatenate([Wg2, Wg3], axis=1)

    row_blk = pl.BlockSpec((BM, N), lambda i: (i, 0))
    bcast = lambda s: pl.BlockSpec(s, lambda i: (0, 0))
    s2 = pl.pallas_call(
        _spmm1_kernel,
        grid=(N // BM,),
        in_specs=[row_blk, bcast((N, GH1)), bcast((GH1, 2 * GH2))],
        out_specs=pl.BlockSpec((BM, 2 * GH2), lambda i: (i, 0)),
        out_shape=jax.ShapeDtypeStruct((N, 2 * GH2), f32),
    )(adj, s1, Wg23)

    out_blk = lambda c: pl.BlockSpec((BM, c), lambda i: (i, 0))
    gcn_mu, gcn_logstd, z, decoded_x = pl.pallas_call(
        _spmm2_dec_kernel,
        grid=(N // BM,),
        in_specs=[row_blk, bcast((N, 2 * GH2)), out_blk(FH2),
                  bcast((LAT, FH1))] + [bcast((1, FH1))] * 5 +
                 [bcast((FH1, D))] + [bcast((1, D))] * 5,
        out_specs=[out_blk(GH2), out_blk(GH2), out_blk(LAT), out_blk(D)],
        out_shape=[jax.ShapeDtypeStruct((N, GH2), f32),
                   jax.ShapeDtypeStruct((N, GH2), f32),
                   jax.ShapeDtypeStruct((N, LAT), f32),
                   jax.ShapeDtypeStruct((N, D), f32)],
    )(adj, s2, feat_x,
      Wd1, _row(bd1), _row(gd1), _row(bed1), _row(rmd1), _row(rvd1),
      Wd2, _row(bd2), _row(gd2), _row(bed2), _row(rmd2), _row(rvd2))

    return (gcn_mu, gcn_logstd, feat_x, gcn_mu, z, decoded_x)


# speedup vs baseline: 1.0192x; 1.0192x over previous
import jax
import jax.numpy as jnp
from jax.experimental import pallas as pl

N = 10000
D = 128
FH1 = 64
FH2 = 32
GH1 = 32
GH2 = 16
LAT = FH2 + GH2
EPS = 1e-3

BM = 400


def _bn(x, g, b, rm, rv):
    return (x - rm) / jnp.sqrt(rv + EPS) * g + b


def _elu(x):
    return jnp.where(x > 0, x, jnp.exp(x) - 1.0)


def _encoder_kernel(x_ref, W1_ref, b1_ref, g1_ref, be1_ref, rm1_ref, rv1_ref,
                    W2_ref, b2_ref, g2_ref, be2_ref, rm2_ref, rv2_ref,
                    Wg1_ref, feat_ref, s1_ref):
    h = jnp.dot(x_ref[...], W1_ref[...], preferred_element_type=jnp.float32)
    h = _elu(_bn(h + b1_ref[...], g1_ref[...], be1_ref[...],
                       rm1_ref[...], rv1_ref[...]))
    f = jnp.dot(h, W2_ref[...], preferred_element_type=jnp.float32)
    f = _elu(_bn(f + b2_ref[...], g2_ref[...], be2_ref[...],
                       rm2_ref[...], rv2_ref[...]))
    feat_ref[...] = f
    s1_ref[...] = jnp.dot(f, Wg1_ref[...], preferred_element_type=jnp.float32)


def _spmm1_kernel(adj_ref, s1_ref, Wg23_ref, s2_ref):
    h1 = jnp.dot(adj_ref[...], s1_ref[...], preferred_element_type=jnp.float32)
    s2_ref[...] = jnp.dot(h1, Wg23_ref[...], preferred_element_type=jnp.float32)


def _spmm2_dec_kernel(adj_ref, s2_ref, feat_ref,
                      Wd1_ref, bd1_ref, gd1_ref, bed1_ref, rmd1_ref, rvd1_ref,
                      Wd2_ref, bd2_ref, gd2_ref, bed2_ref, rmd2_ref, rvd2_ref,
                      mu_ref, ls_ref, z_ref, dec_ref):
    out2 = jnp.dot(adj_ref[...], s2_ref[...], preferred_element_type=jnp.float32)
    mu = out2[:, :GH2]
    mu_ref[...] = mu
    ls_ref[...] = out2[:, GH2:]
    z = jnp.concatenate([feat_ref[...], mu], axis=1)
    z_ref[...] = z
    d = jnp.dot(z, Wd1_ref[...], preferred_element_type=jnp.float32)
    d = _elu(_bn(d + bd1_ref[...], gd1_ref[...], bed1_ref[...],
                       rmd1_ref[...], rvd1_ref[...]))
    dec = jnp.dot(d, Wd2_ref[...], preferred_element_type=jnp.float32)
    dec_ref[...] = jax.nn.relu(_bn(dec + bd2_ref[...], gd2_ref[...], bed2_ref[...],
                                   rmd2_ref[...], rvd2_ref[...]))


def _row(v):
    return v.reshape(1, -1)


def kernel(x, adj, W1, b1, g1, be1, rm1, rv1, W2, b2, g2, be2, rm2, rv2,
           Wg1, Wg2, Wg3,
           Wd1, bd1, gd1, bed1, rmd1, rvd1,
           Wd2, bd2, gd2, bed2, rmd2, rvd2):
    f32 = jnp.float32

    full = lambda s: pl.BlockSpec(s, lambda: (0, 0))
    feat_x, s1 = pl.pallas_call(
        _encoder_kernel,
        grid=(),
        in_specs=[full((N, D)),
                  full((D, FH1))] + [full((1, FH1))] * 5 +
                 [full((FH1, FH2))] + [full((1, FH2))] * 5 +
                 [full((FH2, GH1))],
        out_specs=[full((N, FH2)), full((N, GH1))],
        out_shape=[jax.ShapeDtypeStruct((N, FH2), f32),
                   jax.ShapeDtypeStruct((N, GH1), f32)],
    )(x, W1, _row(b1), _row(g1), _row(be1), _row(rm1), _row(rv1),
      W2, _row(b2), _row(g2), _row(be2), _row(rm2), _row(rv2), Wg1)

    Wg23 = jnp.concatenate([Wg2, Wg3], axis=1)

    row_blk = pl.BlockSpec((BM, N), lambda i: (i, 0))
    bcast = lambda s: pl.BlockSpec(s, lambda i: (0, 0))
    s2 = pl.pallas_call(
        _spmm1_kernel,
        grid=(N // BM,),
        in_specs=[row_blk, bcast((N, GH1)), bcast((GH1, 2 * GH2))],
        out_specs=pl.BlockSpec((BM, 2 * GH2), lambda i: (i, 0)),
        out_shape=jax.ShapeDtypeStruct((N, 2 * GH2), f32),
    )(adj, s1, Wg23)

    out_blk = lambda c: pl.BlockSpec((BM, c), lambda i: (i, 0))
    gcn_mu, gcn_logstd, z, decoded_x = pl.pallas_call(
        _spmm2_dec_kernel,
        grid=(N // BM,),
        in_specs=[row_blk, bcast((N, 2 * GH2)), out_blk(FH2),
                  bcast((LAT, FH1))] + [bcast((1, FH1))] * 5 +
                 [bcast((FH1, D))] + [bcast((1, D))] * 5,
        out_specs=[out_blk(GH2), out_blk(GH2), out_blk(LAT), out_blk(D)],
        out_shape=[jax.ShapeDtypeStruct((N, GH2), f32),
                   jax.ShapeDtypeStruct((N, GH2), f32),
                   jax.ShapeDtypeStruct((N, LAT), f32),
                   jax.ShapeDtypeStruct((N, D), f32)],
    )(adj, s2, feat_x,
      Wd1, _row(bd1), _row(gd1), _row(bed1), _row(rmd1), _row(rvd1),
      Wd2, _row(bd2), _row(gd2), _row(bed2), _row(rmd2), _row(rvd2))

    return (gcn_mu, gcn_logstd, feat_x, gcn_mu, z, decoded_x)


# speedup vs baseline: 1.0655x; 1.0453x over previous
import jax
import jax.numpy as jnp
from jax.experimental import pallas as pl
from jax.experimental.pallas import tpu as pltpu

N = 10000
D = 128
FH1 = 64
FH2 = 32
GH1 = 32
GH2 = 16
LAT = FH2 + GH2
EPS = 1e-3

BM = 400


def _bn(x, g, b, rm, rv):
    return (x - rm) / jnp.sqrt(rv + EPS) * g + b


def _elu(x):
    return jnp.where(x > 0, x, jnp.exp(x) - 1.0)


def _fused_kernel(x_ref, adj_ref,
                  W1_ref, b1_ref, g1_ref, be1_ref, rm1_ref, rv1_ref,
                  W2_ref, b2_ref, g2_ref, be2_ref, rm2_ref, rv2_ref,
                  Wg1_ref, Wg2_ref, Wg3_ref,
                  Wd1_ref, bd1_ref, gd1_ref, bed1_ref, rmd1_ref, rvd1_ref,
                  Wd2_ref, bd2_ref, gd2_ref, bed2_ref, rmd2_ref, rvd2_ref,
                  mu_ref, ls_ref, feat_out_ref, z_ref, dec_ref,
                  feat_sc, s1_sc, s2_sc):
    p = pl.program_id(0)
    i = pl.program_id(1)

    @pl.when((p == 0) & (i == 0))
    def _encoder():
        h = jnp.dot(x_ref[...], W1_ref[...], preferred_element_type=jnp.float32)
        h = _elu(_bn(h + b1_ref[...], g1_ref[...], be1_ref[...],
                     rm1_ref[...], rv1_ref[...]))
        f = jnp.dot(h, W2_ref[...], preferred_element_type=jnp.float32)
        f = _elu(_bn(f + b2_ref[...], g2_ref[...], be2_ref[...],
                     rm2_ref[...], rv2_ref[...]))
        feat_sc[...] = f
        s1_sc[...] = jnp.dot(f, Wg1_ref[...], preferred_element_type=jnp.float32)

    @pl.when(p == 0)
    def _pass1():
        h1 = jnp.dot(adj_ref[...], s1_sc[...], preferred_element_type=jnp.float32)
        s2_sc[pl.ds(i * BM, BM), :] = jnp.concatenate(
            [jnp.dot(h1, Wg2_ref[...], preferred_element_type=jnp.float32),
             jnp.dot(h1, Wg3_ref[...], preferred_element_type=jnp.float32)],
            axis=1)

    @pl.when(p == 1)
    def _pass2():
        out2 = jnp.dot(adj_ref[...], s2_sc[...], preferred_element_type=jnp.float32)
        mu = out2[:, :GH2]
        mu_ref[...] = mu
        ls_ref[...] = out2[:, GH2:]
        feat_blk = feat_sc[pl.ds(i * BM, BM), :]
        feat_out_ref[...] = feat_blk
        z = jnp.concatenate([feat_blk, mu], axis=1)
        z_ref[...] = z
        d = jnp.dot(z, Wd1_ref[...], preferred_element_type=jnp.float32)
        d = _elu(_bn(d + bd1_ref[...], gd1_ref[...], bed1_ref[...],
                     rmd1_ref[...], rvd1_ref[...]))
        dec = jnp.dot(d, Wd2_ref[...], preferred_element_type=jnp.float32)
        dec_ref[...] = jax.nn.relu(_bn(dec + bd2_ref[...], gd2_ref[...],
                                       bed2_ref[...], rmd2_ref[...], rvd2_ref[...]))


def _row(v):
    return v.reshape(1, -1)


def kernel(x, adj, W1, b1, g1, be1, rm1, rv1, W2, b2, g2, be2, rm2, rv2,
           Wg1, Wg2, Wg3,
           Wd1, bd1, gd1, bed1, rmd1, rvd1,
           Wd2, bd2, gd2, bed2, rmd2, rvd2):
    f32 = jnp.float32
    const = lambda s: pl.BlockSpec(s, lambda p, i: (0, 0))
    row_blk = pl.BlockSpec((BM, N), lambda p, i: (i, 0))
    out_blk = lambda c: pl.BlockSpec((BM, c), lambda p, i: (i, 0))

    gcn_mu, gcn_logstd, feat_x, z, decoded_x = pl.pallas_call(
        _fused_kernel,
        grid=(2, N // BM),
        in_specs=[const((N, D)), row_blk,
                  const((D, FH1))] + [const((1, FH1))] * 5 +
                 [const((FH1, FH2))] + [const((1, FH2))] * 5 +
                 [const((FH2, GH1)), const((GH1, GH2)), const((GH1, GH2)),
                  const((LAT, FH1))] + [const((1, FH1))] * 5 +
                 [const((FH1, D))] + [const((1, D))] * 5,
        out_specs=[out_blk(GH2), out_blk(GH2), out_blk(FH2), out_blk(LAT),
                   out_blk(D)],
        out_shape=[jax.ShapeDtypeStruct((N, GH2), f32),
                   jax.ShapeDtypeStruct((N, GH2), f32),
                   jax.ShapeDtypeStruct((N, FH2), f32),
                   jax.ShapeDtypeStruct((N, LAT), f32),
                   jax.ShapeDtypeStruct((N, D), f32)],
        scratch_shapes=[pltpu.VMEM((N, FH2), f32),
                        pltpu.VMEM((N, GH1), f32),
                        pltpu.VMEM((N, 2 * GH2), f32)],
    )(x, adj,
      W1, _row(b1), _row(g1), _row(be1), _row(rm1), _row(rv1),
      W2, _row(b2), _row(g2), _row(be2), _row(rm2), _row(rv2),
      Wg1, Wg2, Wg3,
      Wd1, _row(bd1), _row(gd1), _row(bed1), _row(rmd1), _row(rvd1),
      Wd2, _row(bd2), _row(gd2), _row(bed2), _row(rmd2), _row(rvd2))

    return (gcn_mu, gcn_logstd, feat_x, gcn_mu, z, decoded_x)


# speedup vs baseline: 1.0868x; 1.0200x over previous
import jax
import jax.numpy as jnp
from jax.experimental import pallas as pl
from jax.experimental.pallas import tpu as pltpu

N = 10000
D = 128
FH1 = 64
FH2 = 32
GH1 = 32
GH2 = 16
LAT = FH2 + GH2
EPS = 1e-3

BM = 400


def _bn(x, g, b, rm, rv):
    return (x - rm) / jnp.sqrt(rv + EPS) * g + b


def _elu(x):
    return jnp.where(x > 0, x, jnp.exp(x) - 1.0)


def _fused_kernel(x_ref, adj_ref,
                  W1_ref, b1_ref, g1_ref, be1_ref, rm1_ref, rv1_ref,
                  W2_ref, b2_ref, g2_ref, be2_ref, rm2_ref, rv2_ref,
                  Wg1_ref, Wg2_ref, Wg3_ref,
                  Wd1_ref, bd1_ref, gd1_ref, bed1_ref, rmd1_ref, rvd1_ref,
                  Wd2_ref, bd2_ref, gd2_ref, bed2_ref, rmd2_ref, rvd2_ref,
                  mu_ref, ls_ref, feat_out_ref, z_ref, dec_ref,
                  feat_sc, s1_sc, s2_sc):
    p = pl.program_id(0)
    i = pl.program_id(1)

    @pl.when((p == 0) & (i == 0))
    def _encoder():
        h = jnp.dot(x_ref[...], W1_ref[...], preferred_element_type=jnp.float32)
        h = _elu(_bn(h + b1_ref[...], g1_ref[...], be1_ref[...],
                     rm1_ref[...], rv1_ref[...]))
        f = jnp.dot(h, W2_ref[...], preferred_element_type=jnp.float32)
        f = _elu(_bn(f + b2_ref[...], g2_ref[...], be2_ref[...],
                     rm2_ref[...], rv2_ref[...]))
        feat_sc[...] = f
        s1_sc[...] = jnp.dot(f, Wg1_ref[...], preferred_element_type=jnp.float32)

    @pl.when(p == 0)
    def _pass1():
        h1 = jnp.dot(adj_ref[...], s1_sc[...], preferred_element_type=jnp.float32)
        s2_sc[pl.ds(i * BM, BM), :] = jnp.concatenate(
            [jnp.dot(h1, Wg2_ref[...], preferred_element_type=jnp.float32),
             jnp.dot(h1, Wg3_ref[...], preferred_element_type=jnp.float32)],
            axis=1)

    @pl.when(p == 1)
    def _pass2():
        out2 = jnp.dot(adj_ref[...], s2_sc[...], preferred_element_type=jnp.float32)
        mu = out2[:, :GH2]
        mu_ref[...] = mu
        ls_ref[...] = out2[:, GH2:]
        feat_blk = feat_sc[pl.ds(i * BM, BM), :]
        feat_out_ref[...] = feat_blk
        z = jnp.concatenate([feat_blk, mu], axis=1)
        z_ref[...] = z
        d = jnp.dot(z, Wd1_ref[...], preferred_element_type=jnp.float32)
        d = _elu(_bn(d + bd1_ref[...], gd1_ref[...], bed1_ref[...],
                     rmd1_ref[...], rvd1_ref[...]))
        dec = jnp.dot(d, Wd2_ref[...], preferred_element_type=jnp.float32)
        dec_ref[...] = jax.nn.relu(_bn(dec + bd2_ref[...], gd2_ref[...],
                                       bed2_ref[...], rmd2_ref[...], rvd2_ref[...]))


def _row(v):
    return v.reshape(1, -1)


def kernel(x, adj, W1, b1, g1, be1, rm1, rv1, W2, b2, g2, be2, rm2, rv2,
           Wg1, Wg2, Wg3,
           Wd1, bd1, gd1, bed1, rmd1, rvd1,
           Wd2, bd2, gd2, bed2, rmd2, rvd2):
    f32 = jnp.float32
    const = lambda s: pl.BlockSpec(s, lambda p, i: (0, 0))
    row_blk = pl.BlockSpec((BM, N), lambda p, i: (i, 0))
    out_blk = lambda c: pl.BlockSpec((BM, c), lambda p, i: (p * i, 0))

    gcn_mu, gcn_logstd, feat_x, z, decoded_x = pl.pallas_call(
        _fused_kernel,
        grid=(2, N // BM),
        in_specs=[const((N, D)), row_blk,
                  const((D, FH1))] + [const((1, FH1))] * 5 +
                 [const((FH1, FH2))] + [const((1, FH2))] * 5 +
                 [const((FH2, GH1)), const((GH1, GH2)), const((GH1, GH2)),
                  const((LAT, FH1))] + [const((1, FH1))] * 5 +
                 [const((FH1, D))] + [const((1, D))] * 5,
        out_specs=[out_blk(GH2), out_blk(GH2), out_blk(FH2), out_blk(LAT),
                   out_blk(D)],
        out_shape=[jax.ShapeDtypeStruct((N, GH2), f32),
                   jax.ShapeDtypeStruct((N, GH2), f32),
                   jax.ShapeDtypeStruct((N, FH2), f32),
                   jax.ShapeDtypeStruct((N, LAT), f32),
                   jax.ShapeDtypeStruct((N, D), f32)],
        scratch_shapes=[pltpu.VMEM((N, FH2), f32),
                        pltpu.VMEM((N, GH1), f32),
                        pltpu.VMEM((N, 2 * GH2), f32)],
    )(x, adj,
      W1, _row(b1), _row(g1), _row(be1), _row(rm1), _row(rv1),
      W2, _row(b2), _row(g2), _row(be2), _row(rm2), _row(rv2),
      Wg1, Wg2, Wg3,
      Wd1, _row(bd1), _row(gd1), _row(bed1), _row(rmd1), _row(rvd1),
      Wd2, _row(bd2), _row(gd2), _row(bed2), _row(rmd2), _row(rvd2))

    return (gcn_mu, gcn_logstd, feat_x, gcn_mu, z, decoded_x)


# speedup vs baseline: 1.0920x; 1.0048x over previous
import jax
import jax.numpy as jnp
from jax.experimental import pallas as pl
from jax.experimental.pallas import tpu as pltpu

N = 10000
D = 128
FH1 = 64
FH2 = 32
GH1 = 32
GH2 = 16
LAT = FH2 + GH2
EPS = 1e-3

BMC = 200
NCH = N // BMC
NBUF = 4
TOT = 2 * NCH


def _bn(x, g, b, rm, rv):
    return (x - rm) / jnp.sqrt(rv + EPS) * g + b


def _elu(x):
    return jnp.where(x > 0, x, jnp.exp(x) - 1.0)


def _fused_kernel(x_ref, adj_hbm,
                  W1_ref, b1_ref, g1_ref, be1_ref, rm1_ref, rv1_ref,
                  W2_ref, b2_ref, g2_ref, be2_ref, rm2_ref, rv2_ref,
                  Wg1_ref, Wg2_ref, Wg3_ref,
                  Wd1_ref, bd1_ref, gd1_ref, bed1_ref, rmd1_ref, rvd1_ref,
                  Wd2_ref, bd2_ref, gd2_ref, bed2_ref, rmd2_ref, rvd2_ref,
                  big_ref, dec_ref,
                  bufs, s1_sc, s2_sc, sems):

    def start_copy(step, slot):
        c = jax.lax.rem(step, NCH)
        pltpu.make_async_copy(adj_hbm.at[pl.ds(c * BMC, BMC), :],
                              bufs.at[slot], sems.at[slot]).start()

    def wait_copy(slot):
        pltpu.make_async_copy(adj_hbm.at[pl.ds(0, BMC), :],
                              bufs.at[slot], sems.at[slot]).wait()

    for k in range(NBUF):
        start_copy(k, k)

    h = jnp.dot(x_ref[...], W1_ref[...], preferred_element_type=jnp.float32)
    h = _elu(_bn(h + b1_ref[...], g1_ref[...], be1_ref[...],
                 rm1_ref[...], rv1_ref[...]))
    f = jnp.dot(h, W2_ref[...], preferred_element_type=jnp.float32)
    f = _elu(_bn(f + b2_ref[...], g2_ref[...], be2_ref[...],
                 rm2_ref[...], rv2_ref[...]))
    big_ref[:, :FH2] = f
    s1_sc[...] = jnp.dot(f, Wg1_ref[...], preferred_element_type=jnp.float32)

    def process(s, slot):
        c = jax.lax.rem(s, NCH)
        p = s // NCH
        rows = pl.ds(c * BMC, BMC)
        wait_copy(slot)
        a_ref = bufs.at[slot]

        @pl.when(p == 0)
        def _pass1():
            h1 = jnp.dot(a_ref[...], s1_sc[...],
                         preferred_element_type=jnp.float32)
            s2_sc[rows, :] = jnp.concatenate(
                [jnp.dot(h1, Wg2_ref[...], preferred_element_type=jnp.float32),
                 jnp.dot(h1, Wg3_ref[...], preferred_element_type=jnp.float32)],
                axis=1)

        @pl.when(p == 1)
        def _pass2():
            out2 = jnp.dot(a_ref[...], s2_sc[...],
                           preferred_element_type=jnp.float32)
            mu = out2[:, :GH2]
            big_ref[rows, FH2:FH2 + GH2] = mu
            big_ref[rows, FH2 + GH2:] = out2[:, GH2:]
            z = jnp.concatenate([big_ref[rows, :FH2], mu], axis=1)
            d = jnp.dot(z, Wd1_ref[...], preferred_element_type=jnp.float32)
            d = _elu(_bn(d + bd1_ref[...], gd1_ref[...], bed1_ref[...],
                         rmd1_ref[...], rvd1_ref[...]))
            dec = jnp.dot(d, Wd2_ref[...], preferred_element_type=jnp.float32)
            dec_ref[rows, :] = jax.nn.relu(
                _bn(dec + bd2_ref[...], gd2_ref[...], bed2_ref[...],
                    rmd2_ref[...], rvd2_ref[...]))

        @pl.when(s + NBUF < TOT)
        def _next():
            start_copy(s + NBUF, slot)

    def body(j, _):
        base = j * NBUF
        for k in range(NBUF):
            process(base + k, k)
        return 0

    jax.lax.fori_loop(0, TOT // NBUF, body, 0)


def _row(v):
    return v.reshape(1, -1)


def kernel(x, adj, W1, b1, g1, be1, rm1, rv1, W2, b2, g2, be2, rm2, rv2,
           Wg1, Wg2, Wg3,
           Wd1, bd1, gd1, bed1, rmd1, rvd1,
           Wd2, bd2, gd2, bed2, rmd2, rvd2):
    f32 = jnp.float32
    vmem = pl.BlockSpec(memory_space=pltpu.VMEM)

    big, decoded_x = pl.pallas_call(
        _fused_kernel,
        in_specs=[vmem, pl.BlockSpec(memory_space=pl.ANY)] + [vmem] * 27,
        out_specs=[vmem] * 2,
        out_shape=[jax.ShapeDtypeStruct((N, FH2 + 2 * GH2), f32),
                   jax.ShapeDtypeStruct((N, D), f32)],
        scratch_shapes=[pltpu.VMEM((NBUF, BMC, N), f32),
                        pltpu.VMEM((N, GH1), f32),
                        pltpu.VMEM((N, 2 * GH2), f32),
                        pltpu.SemaphoreType.DMA((NBUF,))],
        compiler_params=pltpu.CompilerParams(
            vmem_limit_bytes=64 * 1024 * 1024),
    )(x, adj,
      W1, _row(b1), _row(g1), _row(be1), _row(rm1), _row(rv1),
      W2, _row(b2), _row(g2), _row(be2), _row(rm2), _row(rv2),
      Wg1, Wg2, Wg3,
      Wd1, _row(bd1), _row(gd1), _row(bed1), _row(rmd1), _row(rvd1),
      Wd2, _row(bd2), _row(gd2), _row(bed2), _row(rmd2), _row(rvd2))

    feat_x = big[:, :FH2]
    gcn_mu = big[:, FH2:FH2 + GH2]
    gcn_logstd = big[:, FH2 + GH2:]
    z = big[:, :LAT]
    return (gcn_mu, gcn_logstd, feat_x, gcn_mu, z, decoded_x)
